# Initial kernel scaffold; baseline (speedup 1.0000x reference)
import jax
import jax.numpy as jnp
from jax.experimental import pallas as pl


def kernel(x, Wp, bp, codebook, pos_emb):
    raise NotImplementedError("write your pallas kernel here")


# speedup vs baseline: 1.4075x; 1.4075x over previous
import functools

import jax
import jax.numpy as jnp
from jax import lax
from jax.experimental import pallas as pl
from jax.experimental.pallas import tpu as pltpu
from jax.experimental.pallas import tpu_sc as plsc

_MT = 256
_NC, _NS, _L = 2, 16, 16
_CH = 128


def _tc_body(x_ref, wpt_ref, bp_ref, cbt_ref, idx_ref, loss_ref, e2_ref):
    i = pl.program_id(0)
    K = cbt_ref.shape[1]

    @pl.when(i == 0)
    def _init():
        cbt = cbt_ref[...]
        e2_ref[...] = jnp.sum(cbt * cbt, axis=0, keepdims=True)
        loss_ref[...] = jnp.zeros((1, 1), jnp.float32)

    z = jnp.dot(x_ref[...], wpt_ref[...],
                preferred_element_type=jnp.float32) + bp_ref[...]
    dots = jnp.dot(z, cbt_ref[...],
                   preferred_element_type=jnp.float32)
    z2 = jnp.sum(z * z, axis=1, keepdims=True)
    dist = z2 - 2.0 * dots + e2_ref[...]
    mind = jnp.min(dist, axis=1, keepdims=True)
    ids = lax.broadcasted_iota(jnp.int32, dist.shape, 1)
    idx = jnp.min(jnp.where(dist == mind, ids, K), axis=1).astype(jnp.int32)
    idx_ref[0, 0, :] = idx
    loss_ref[...] = loss_ref[...] + jnp.sum(mind).reshape(1, 1)


def _tc_call(xf, wpt, bp2, cbt):
    M, F = xf.shape
    D, K = cbt.shape
    grid = (M // _MT,)
    return pl.pallas_call(
        _tc_body,
        grid=grid,
        in_specs=[
            pl.BlockSpec((_MT, F), lambda i: (i, 0)),
            pl.BlockSpec((F, D), lambda i: (0, 0)),
            pl.BlockSpec((1, D), lambda i: (0, 0)),
            pl.BlockSpec((D, K), lambda i: (0, 0)),
        ],
        out_specs=[
            pl.BlockSpec((1, 1, _MT), lambda i: (i, 0, 0)),
            pl.BlockSpec((1, 1), lambda i: (0, 0)),
        ],
        out_shape=[
            jax.ShapeDtypeStruct((M // _MT, 1, _MT), jnp.int32),
            jax.ShapeDtypeStruct((1, 1), jnp.float32),
        ],
        scratch_shapes=[pltpu.VMEM((1, K), jnp.float32)],
        compiler_params=pltpu.CompilerParams(
            dimension_semantics=("arbitrary",)),
    )(xf, wpt, bp2, cbt)


def _make_sc_gather(M, N, D):
    NW = _NC * _NS
    bpw = M // NW
    nrounds = bpw // _CH
    mesh = plsc.VectorSubcoreMesh(core_axis_name="c", subcore_axis_name="s")

    @functools.partial(
        pl.kernel, mesh=mesh,
        out_type=jax.ShapeDtypeStruct((M, D), jnp.float32),
        scratch_types=[
            pltpu.VMEM((_CH,), jnp.int32),
            pltpu.VMEM((_CH, D), jnp.float32),
            pltpu.VMEM((_CH, D), jnp.float32),
            pltpu.SemaphoreType.DMA,
        ],
    )
    def sc_gather(cb_hbm, idx_hbm, pos_hbm, out_hbm, idx_v, rows_v, pos_v, sem):
        wid = lax.axis_index("s") * _NC + lax.axis_index("c")
        for r in range(nrounds):
            base = wid * bpw + r * _CH
            pbase = lax.rem(base, N)
            pltpu.sync_copy(idx_hbm.at[pl.ds(base, _CH)], idx_v)
            cp = pltpu.async_copy(cb_hbm.at[idx_v], rows_v, sem)
            pltpu.sync_copy(pos_hbm.at[pl.ds(pbase, _CH)], pos_v)
            cp.wait()

            def body(i, c):
                for j in range(D // _L):
                    sl = pl.ds(j * _L, _L)
                    rows_v[i, sl] = rows_v[i, sl] + pos_v[i, sl]
                return c

            lax.fori_loop(0, _CH, body, 0)
            pltpu.sync_copy(rows_v, out_hbm.at[pl.ds(base, _CH)])

    return sc_gather


def kernel(x, Wp, bp, codebook, pos_emb):
    B, N, F = x.shape
    D = Wp.shape[0]
    K = codebook.shape[0]
    M = B * N

    xf = x.reshape(M, F)
    wpt = Wp.T
    bp2 = bp.reshape(1, D)
    cbt = codebook.T
    pos2 = pos_emb.reshape(N, D)

    idx3, loss_sum = _tc_call(xf, wpt, bp2, cbt)
    idx_flat = idx3.reshape(M)

    out_flat = _make_sc_gather(M, N, D)(codebook, idx_flat, pos2)
    out = out_flat.reshape(B, N, D)
    commit_loss = loss_sum[0, 0] / jnp.float32(M * D)
    return (out, commit_loss)
